# speedup vs baseline: 2.9293x; 2.9293x over previous
import functools

import jax
import jax.numpy as jnp
from jax.experimental import pallas as pl


def _matmul_bias_kernel(x_ref, w_ref, b_ref, o_ref, *, relu):
    acc = jnp.dot(x_ref[...], w_ref[...], preferred_element_type=jnp.float32)
    acc = acc + b_ref[...]
    if relu:
        acc = jnp.maximum(acc, 0.0)
    o_ref[...] = acc


def _matmul_bias(x, w, b, relu=False, block=2048):
    n, d = x.shape
    nb = pl.cdiv(n, block)
    return pl.pallas_call(
        functools.partial(_matmul_bias_kernel, relu=relu),
        grid=(nb,),
        in_specs=[
            pl.BlockSpec((block, d), lambda i: (i, 0)),
            pl.BlockSpec((d, d), lambda i: (0, 0)),
            pl.BlockSpec((d,), lambda i: (0,)),
        ],
        out_specs=pl.BlockSpec((block, d), lambda i: (i, 0)),
        out_shape=jax.ShapeDtypeStruct((n, d), jnp.float32),
    )(x, w, b)


def _gcn_conv(x, edge_index, W, b, num_nodes):
    y = _matmul_bias(x, W, jnp.zeros_like(b))
    row = edge_index[0]
    col = edge_index[1]
    deg = jnp.ones((num_nodes,), dtype=x.dtype).at[col].add(1.0)
    dinv = jax.lax.rsqrt(deg)
    z = y * dinv[:, None]
    acc = z.at[col].add(z[row])
    return acc * dinv[:, None] + b


def kernel(x, edge_index, line_graph_edge_index, W1, b1, W2, b2,
           W_b2a, b_b2a, W_a2b, b_a2b):
    num_atoms = x.shape[0]
    edge_index = edge_index.astype(jnp.int32)
    line_graph_edge_index = line_graph_edge_index.astype(jnp.int32)
    row = edge_index[0]
    col = edge_index[1]

    x_atom = jax.nn.relu(_gcn_conv(x, edge_index, W1, b1, num_atoms))
    x_bond = (x_atom[row] + x_atom[col]) * 0.5
    num_bonds = x_bond.shape[0]
    x_bond = jax.nn.relu(_gcn_conv(x_bond, line_graph_edge_index, W2, b2, num_bonds))

    m = _matmul_bias(x_bond, W_b2a, b_b2a)
    atom_message = jnp.zeros((num_atoms, m.shape[1]), dtype=x_atom.dtype)
    atom_message = atom_message.at[row].add(m)
    atom_message = atom_message.at[col].add(m)
    x_atom = x_atom + atom_message

    t = _matmul_bias(x_atom, W_a2b, b_a2b, relu=True)
    x_bond = x_bond + (t[row] + t[col]) * 0.5
    return (x_atom, x_bond)


# speedup vs baseline: 7.4095x; 2.5294x over previous
import functools

import jax
import jax.numpy as jnp
from jax import lax
from jax.experimental import pallas as pl
from jax.experimental.pallas import tpu as pltpu
from jax.experimental.pallas import tpu_sc as plsc

N = 10000
NPAD = 10240
E = 320000
ELG = 640000
D = 128
EPAD = 327680

NC, NS, L = 2, 16, 16
NW = NC * NS
EB = 512
NB_E = E // EB

@functools.cache
def _mesh():
    return plsc.VectorSubcoreMesh(core_axis_name="c", subcore_axis_name="s")


def _wid():
    return lax.axis_index("s") * NC + lax.axis_index("c")


def _nbatches_arr(w, total_batches):
    return (total_batches - w + NW - 1) // NW


def _fill(ref, value):
    n = ref.shape[0]
    for j in range(n // L):
        ref[pl.ds(j * L, L)] = jnp.full((L,), value, ref.dtype)


@functools.cache
def _mk_k1():
    return pl.kernel(
        _k1_degrees_body,
        out_type=[jax.ShapeDtypeStruct((NC, NPAD), jnp.float32),
                  jax.ShapeDtypeStruct((NC, EPAD), jnp.float32)],
        mesh=_mesh(),
        scratch_types=[
            pltpu.VMEM((EB,), jnp.int32),
            pltpu.VMEM((EB,), jnp.float32),
            pltpu.VMEM((EPAD // NS,), jnp.float32),
            pltpu.VMEM_SHARED((NPAD,), jnp.float32),
            pltpu.VMEM_SHARED((EPAD,), jnp.float32),
        ],
    )


def _k1_degrees_body(col1_hbm, lgcol_hbm, d1_hbm, d2_hbm,
                     idx_v, ones_v, zero_v, h1_sh, h2_sh):
    w = _wid()
    core = lax.axis_index("c")
    sid = lax.axis_index("s")
    _fill(ones_v, 1.0)
    zero_v[...] = jnp.zeros_like(zero_v)
    pltpu.sync_copy(zero_v.at[pl.ds(0, NPAD // NS)],
                    h1_sh.at[pl.ds(pl.multiple_of(sid * (NPAD // NS), 128), NPAD // NS)])
    pltpu.sync_copy(zero_v, h2_sh.at[pl.ds(pl.multiple_of(sid * (EPAD // NS), 128), EPAD // NS)])
    plsc.subcore_barrier()

    def body1(k, _):
        base = pl.multiple_of((w + k * NW) * EB, EB)
        pltpu.sync_copy(col1_hbm.at[pl.ds(base, EB)], idx_v)
        pltpu.sync_copy(ones_v, h1_sh.at[idx_v], add=True)
        return 0

    lax.fori_loop(0, _nbatches_arr(w, NB_E), body1, 0)

    def body2(k, _):
        base = pl.multiple_of((w + k * NW) * EB, EB)
        pltpu.sync_copy(lgcol_hbm.at[pl.ds(base, EB)], idx_v)
        pltpu.sync_copy(ones_v, h2_sh.at[idx_v], add=True)
        return 0

    lax.fori_loop(0, _nbatches_arr(w, ELG // EB), body2, 0)
    plsc.subcore_barrier()
    pltpu.sync_copy(h1_sh.at[pl.ds(pl.multiple_of(sid * (NPAD // NS), 128), NPAD // NS)],
                    d1_hbm.at[core].at[pl.ds(sid * (NPAD // NS), NPAD // NS)])
    off2 = pl.multiple_of(sid * (EPAD // NS), 128)
    pltpu.sync_copy(h2_sh.at[pl.ds(off2, EPAD // NS)],
                    d2_hbm.at[core].at[pl.ds(off2, EPAD // NS)])

def _mk_edge_pass(mode):
    eb = 256 if mode in ("gcn", "scat2") else 512
    scratch = [
        pltpu.VMEM((eb,), jnp.int32),
        pltpu.VMEM((eb,), jnp.int32),
        pltpu.VMEM((eb, D), jnp.float32),
        pltpu.SemaphoreType.DMA,
    ]
    if mode in ("gcn", "scat2"):
        scratch.append(pltpu.VMEM_SHARED((NPAD, D), jnp.float32))
        out_type = jax.ShapeDtypeStruct((NC, NPAD, D), jnp.float32)
    else:
        out_type = jax.ShapeDtypeStruct((E, D), jnp.float32)

    def body(*refs):
        if mode == "gcn":
            table_hbm, row_hbm, col_hbm, out_hbm, ri, ci, buf, sem, acc_sh = refs
        elif mode == "scat2":
            vals_hbm, row_hbm, col_hbm, out_hbm, ri, ci, buf, sem, acc_sh = refs
        elif mode == "pair":
            table_hbm, row_hbm, col_hbm, out_hbm, ri, ci, buf, sem = refs
        else:
            base_hbm, table_hbm, row_hbm, col_hbm, out_hbm, ri, ci, buf, sem = refs
        w = _wid()
        core = lax.axis_index("c")
        sid = lax.axis_index("s")

        if mode in ("gcn", "scat2"):
            buf[...] = jnp.zeros_like(buf)
            rps = NPAD // NS
            off = 0
            while off < rps:
                sz = min(eb, rps - off)
                pltpu.sync_copy(
                    buf.at[pl.ds(0, sz)],
                    acc_sh.at[pl.ds(pl.multiple_of(sid * rps + off, 128), sz)])
                off += sz
            plsc.subcore_barrier()

        def step(k, _):
            base = pl.multiple_of((w + k * NW) * eb, eb)
            pltpu.sync_copy(row_hbm.at[pl.ds(base, eb)], ri)
            pltpu.sync_copy(col_hbm.at[pl.ds(base, eb)], ci)
            if mode == "gcn":
                pltpu.async_copy(table_hbm.at[ri], buf, sem).wait()
                pltpu.sync_copy(buf, acc_sh.at[ci], add=True)
            elif mode == "scat2":
                pltpu.sync_copy(vals_hbm.at[pl.ds(base, eb)], buf)
                pltpu.sync_copy(buf, acc_sh.at[ri], add=True)
                pltpu.sync_copy(buf, acc_sh.at[ci], add=True)
            elif mode == "pair":
                pltpu.async_copy(table_hbm.at[ri], buf, sem).wait()
                pltpu.async_copy(table_hbm.at[ci], buf, sem, add=True).wait()
                pltpu.sync_copy(buf, out_hbm.at[pl.ds(base, eb)])
            else:
                pltpu.sync_copy(base_hbm.at[pl.ds(base, eb)], buf)
                pltpu.async_copy(table_hbm.at[ri], buf, sem, add=True).wait()
                pltpu.async_copy(table_hbm.at[ci], buf, sem, add=True).wait()
                pltpu.sync_copy(buf, out_hbm.at[pl.ds(base, eb)])
            return 0

        lax.fori_loop(0, _nbatches_arr(w, E // eb), step, 0)

        if mode in ("gcn", "scat2"):
            plsc.subcore_barrier()
            rps = NPAD // NS
            off = 0
            while off < rps:
                sz = min(eb, rps - off)
                o = pl.multiple_of(sid * rps + off, 128)
                pltpu.sync_copy(acc_sh.at[pl.ds(o, sz)],
                                out_hbm.at[core].at[pl.ds(o, sz)])
                off += sz

    return pl.kernel(body, out_type=out_type, mesh=_mesh(),
                     scratch_types=scratch)


_mk_edge_pass = functools.cache(_mk_edge_pass)


def _dinv_body(d_ref, o_ref):
    o_ref[...] = lax.rsqrt(1.0 + d_ref[0, :] + d_ref[1, :])


def _dinv(dp):
    n = dp.shape[1]
    return pl.pallas_call(
        _dinv_body,
        out_shape=jax.ShapeDtypeStruct((n,), jnp.float32),
    )(dp)

def _k2_body(x_ref, w_ref, dinv_ref, z1_ref):
    y = jnp.dot(x_ref[...], w_ref[...], preferred_element_type=jnp.float32)
    z1_ref[...] = y * dinv_ref[...]


def _k4_body(p0_ref, p1_ref, z1_ref, dinv_ref, b_ref, h_ref, hp_ref):
    out = (p0_ref[...] + p1_ref[...] + z1_ref[...]) * dinv_ref[...] + b_ref[...]
    h = jnp.maximum(out, 0.0)
    h_ref[...] = h
    hp_ref[...] = h * 0.5


def _k6_body(xb_ref, w_ref, dinv_ref, z2_ref):
    y = jnp.dot(xb_ref[...], w_ref[...], preferred_element_type=jnp.float32)
    z2_ref[...] = y * dinv_ref[...]


def _k8_body(acc_ref, z2_ref, dinv_ref, b2_ref, wm_ref, bm_ref,
             hb_ref, m_ref):
    hb = jnp.maximum((acc_ref[...] + z2_ref[...]) * dinv_ref[...] + b2_ref[...],
                     0.0)
    hb_ref[...] = hb
    m_ref[...] = jnp.dot(hb, wm_ref[...],
                         preferred_element_type=jnp.float32) + bm_ref[...]


def _k10_body(h_ref, q0_ref, q1_ref, w_ref, b_ref, xa_ref, tp_ref):
    xa = h_ref[...] + q0_ref[...] + q1_ref[...]
    xa_ref[...] = xa
    u = jnp.dot(xa, w_ref[...], preferred_element_type=jnp.float32) + b_ref[...]
    tp_ref[...] = jnp.maximum(u * 0.5, 0.0)


def _row_spec(blk):
    return pl.BlockSpec((blk, D), lambda i: (i, 0))


def _col_spec(blk):
    return pl.BlockSpec((blk, 1), lambda i: (i, 0))


def _mat_spec():
    return pl.BlockSpec((D, D), lambda i: (0, 0))


def _bias_spec():
    return pl.BlockSpec((D,), lambda i: (0,))


def kernel(x, edge_index, line_graph_edge_index, W1, b1, W2, b2,
           W_b2a, b_b2a, W_a2b, b_a2b):
    edge_index = edge_index.astype(jnp.int32)
    line_graph_edge_index = line_graph_edge_index.astype(jnp.int32)
    row = edge_index[0]
    col = edge_index[1]
    lgrow = line_graph_edge_index[0]
    lgcol = line_graph_edge_index[1]

    d1p, d2p = _mk_k1()(col, lgcol)
    dinv1 = _dinv(d1p)[:N].reshape(N, 1)
    dinv2 = _dinv(d2p)[:E].reshape(E, 1)

    blk = 2000
    z1 = pl.pallas_call(
        _k2_body,
        grid=(N // blk,),
        in_specs=[_row_spec(blk), _mat_spec(), _col_spec(blk)],
        out_specs=_row_spec(blk),
        out_shape=jax.ShapeDtypeStruct((N, D), jnp.float32),
    )(x, W1, dinv1)

    acc1p = _mk_edge_pass("gcn")(z1, row, col)
    p0 = acc1p[0, :N]
    p1 = acc1p[1, :N]

    h, hp = pl.pallas_call(
        _k4_body,
        grid=(N // blk,),
        in_specs=[_row_spec(blk), _row_spec(blk), _row_spec(blk),
                  _col_spec(blk), _bias_spec()],
        out_specs=[_row_spec(blk), _row_spec(blk)],
        out_shape=[jax.ShapeDtypeStruct((N, D), jnp.float32),
                   jax.ShapeDtypeStruct((N, D), jnp.float32)],
    )(p0, p1, z1, dinv1, b1)

    xb0 = _mk_edge_pass("pair")(hp, row, col)

    eblk = 2560
    z2 = pl.pallas_call(
        _k6_body,
        grid=(E // eblk,),
        in_specs=[_row_spec(eblk), _mat_spec(), _col_spec(eblk)],
        out_specs=_row_spec(eblk),
        out_shape=jax.ShapeDtypeStruct((E, D), jnp.float32),
    )(xb0, W2, dinv2)

    acc2 = jnp.zeros((E, D), jnp.float32).at[lgcol].add(z2[lgrow])

    hb, m = pl.pallas_call(
        _k8_body,
        grid=(E // eblk,),
        in_specs=[_row_spec(eblk), _row_spec(eblk), _col_spec(eblk),
                  _bias_spec(), _mat_spec(), _bias_spec()],
        out_specs=[_row_spec(eblk), _row_spec(eblk)],
        out_shape=[jax.ShapeDtypeStruct((E, D), jnp.float32),
                   jax.ShapeDtypeStruct((E, D), jnp.float32)],
    )(acc2, z2, dinv2, b2, W_b2a, b_b2a)

    accmp = _mk_edge_pass("scat2")(m, row, col)
    q0 = accmp[0, :N]
    q1 = accmp[1, :N]

    x_atom, tp = pl.pallas_call(
        _k10_body,
        grid=(N // blk,),
        in_specs=[_row_spec(blk), _row_spec(blk), _row_spec(blk),
                  _mat_spec(), _bias_spec()],
        out_specs=[_row_spec(blk), _row_spec(blk)],
        out_shape=[jax.ShapeDtypeStruct((N, D), jnp.float32),
                   jax.ShapeDtypeStruct((N, D), jnp.float32)],
    )(h, q0, q1, W_a2b, b_a2b)

    x_bond = _mk_edge_pass("final")(hb, tp, row, col)

    return (x_atom, x_bond)
